# Initial kernel scaffold; baseline (speedup 1.0000x reference)
import functools

import jax
import jax.numpy as jnp
from jax import lax
from jax.experimental import pallas as pl
from jax.experimental.pallas import tpu as pltpu
from jax.experimental.pallas import tpu_sc as plsc

N = 50000
E = 800000
H = 64
G = 64
EPS = 1e-5

N_PAD = 50176
DUMMY = N
E_ROWS = 6400
BLK = 1024
NUM_BLK = N_PAD // BLK
SUBSL = N_PAD // 16
SUP = 8


def _sc_agg_scalar(x_pad, src2d, dst2d, zeros1):
    mesh = plsc.VectorSubcoreMesh(core_axis_name="c", subcore_axis_name="s")

    @functools.partial(
        pl.kernel,
        out_type=jax.ShapeDtypeStruct((2, N_PAD, 1), jnp.float32),
        mesh=mesh,
        scratch_types=[
            pltpu.VMEM_SHARED((N_PAD, 1), jnp.float32),
            pltpu.VMEM((SUP, 128), jnp.int32),
            pltpu.VMEM((SUP, 128), jnp.int32),
            pltpu.VMEM((SUP * 128, 1), jnp.float32),
            pltpu.SemaphoreType.DMA,
        ],
    )
    def k(x_hbm, src_hbm, dst_hbm, zz_hbm, out_hbm, acc, sidx, didx, rows, sem):
        c = lax.axis_index("c")
        s = lax.axis_index("s")
        w0 = s * SUBSL
        pltpu.sync_copy(zz_hbm.at[pl.ds(w0, SUBSL)], acc.at[pl.ds(w0, SUBSL)])
        plsc.subcore_barrier()

        def body(i, carry):
            r0 = c * 3200 + s * 200 + i * SUP
            pltpu.sync_copy(src_hbm.at[pl.ds(r0, SUP)], sidx)
            pltpu.sync_copy(dst_hbm.at[pl.ds(r0, SUP)], didx)
            cps = [
                pltpu.async_copy(x_hbm.at[sidx.at[j]],
                                 rows.at[pl.ds(j * 128, 128)], sem)
                for j in range(SUP)
            ]
            for cp in cps:
                cp.wait()
            for j in range(SUP):
                pltpu.sync_copy(rows.at[pl.ds(j * 128, 128)],
                                acc.at[didx.at[j]], add=True)
            return carry

        lax.fori_loop(0, 25, body, 0)
        plsc.subcore_barrier()

        @pl.when(c == 0)
        def _():
            pltpu.sync_copy(acc.at[pl.ds(w0, SUBSL)],
                            out_hbm.at[0].at[pl.ds(w0, SUBSL)])

        @pl.when(c == 1)
        def _():
            pltpu.sync_copy(acc.at[pl.ds(w0, SUBSL)],
                            out_hbm.at[1].at[pl.ds(w0, SUBSL)])

    return k(x_pad, src2d, dst2d, zeros1)


def _sc_agg_h(h_split, src2d, dst2d, zeros32):
    mesh = plsc.VectorSubcoreMesh(core_axis_name="c", subcore_axis_name="s")

    @functools.partial(
        pl.kernel,
        out_type=jax.ShapeDtypeStruct((2, N_PAD, 32), jnp.float32),
        mesh=mesh,
        scratch_types=[
            pltpu.VMEM_SHARED((N_PAD, 32), jnp.float32),
            pltpu.VMEM((SUP, 128), jnp.int32),
            pltpu.VMEM((SUP, 128), jnp.int32),
            pltpu.VMEM((SUP * 128, 32), jnp.float32),
            pltpu.SemaphoreType.DMA,
        ],
    )
    def k(h_hbm, src_hbm, dst_hbm, zz_hbm, out_hbm, acc, sidx, didx, rows, sem):
        c = lax.axis_index("c")
        s = lax.axis_index("s")
        w0 = s * SUBSL
        pltpu.sync_copy(zz_hbm.at[pl.ds(w0, SUBSL)], acc.at[pl.ds(w0, SUBSL)])
        plsc.subcore_barrier()

        def make_body(h_view):
            def body(i, carry):
                r0 = s * 400 + i * SUP
                pltpu.sync_copy(src_hbm.at[pl.ds(r0, SUP)], sidx)
                pltpu.sync_copy(dst_hbm.at[pl.ds(r0, SUP)], didx)
                cps = [
                    pltpu.async_copy(h_view.at[sidx.at[j]],
                                     rows.at[pl.ds(j * 128, 128)], sem)
                    for j in range(SUP)
                ]
                for cp in cps:
                    cp.wait()
                for j in range(SUP):
                    pltpu.sync_copy(rows.at[pl.ds(j * 128, 128)],
                                    acc.at[didx.at[j]], add=True)
                return carry
            return body

        @pl.when(c == 0)
        def _():
            lax.fori_loop(0, 50, make_body(h_hbm.at[0]), 0)

        @pl.when(c == 1)
        def _():
            lax.fori_loop(0, 50, make_body(h_hbm.at[1]), 0)

        plsc.subcore_barrier()

        @pl.when(c == 0)
        def _():
            pltpu.sync_copy(acc.at[pl.ds(w0, SUBSL)],
                            out_hbm.at[0].at[pl.ds(w0, SUBSL)])

        @pl.when(c == 1)
        def _():
            pltpu.sync_copy(acc.at[pl.ds(w0, SUBSL)],
                            out_hbm.at[1].at[pl.ds(w0, SUBSL)])

    return k(h_split, src2d, dst2d, zeros32)


def _tc1a_body(aggp_ref, x_ref, a_ref, parts_ref):
    i = pl.program_id(0)
    a = aggp_ref[0] + aggp_ref[1]
    a_ref[...] = a
    x = x_ref[...]
    row = i * BLK + lax.broadcasted_iota(jnp.int32, (BLK, 1), 0)
    mask = row < N
    am = jnp.where(mask, a, 0.0)
    xm = jnp.where(mask, x, 0.0)

    @pl.when(i == 0)
    def _():
        parts_ref[...] = jnp.zeros_like(parts_ref)

    parts_ref[:, 0:1] += am
    parts_ref[:, 1:2] += am * am
    parts_ref[:, 2:3] += xm
    parts_ref[:, 3:4] += xm * xm
    parts_ref[:, 4:5] += am * xm


def _tc1b_body(a_ref, x_ref, parts_ref, al_ref, ber_ref, bb_ref, g_ref,
               bt_ref, h_ref):
    p = parts_ref[...]
    ninv = 1.0 / N
    ma = jnp.sum(p[:, 0]) * ninv
    saa = jnp.sum(p[:, 1]) * ninv
    mx = jnp.sum(p[:, 2]) * ninv
    sxx = jnp.sum(p[:, 3]) * ninv
    sax = jnp.sum(p[:, 4]) * ninv
    va = saa - ma * ma
    vx = sxx - mx * mx
    cax = sax - ma * mx
    al = al_ref[...]
    ber = ber_ref[...]
    bb = bb_ref[...]
    m = ma * al + mx * ber + bb
    v = al * al * va + 2.0 * al * ber * cax + ber * ber * vx
    scale = g_ref[...] * lax.rsqrt(v + EPS)
    A = al * scale
    B = ber * scale
    C = (bb - m) * scale + bt_ref[...]
    a = a_ref[...]
    x = x_ref[...]
    h = a[None, :, :] * A[:, None, :] + x[None, :, :] * B[:, None, :] \
        + C[:, None, :]
    h_ref[...] = jnp.maximum(h, 0.0)


def _tc_layer_body(agg_ref, h_ref, wr_ref, wo_ref, b_ref, z_ref, parts_ref):
    i = pl.program_id(0)
    a = jnp.concatenate([agg_ref[0], agg_ref[1]], axis=1)
    h = jnp.concatenate([h_ref[0], h_ref[1]], axis=1)
    z = (jnp.dot(a, wr_ref[...], preferred_element_type=jnp.float32)
         + jnp.dot(h, wo_ref[...], preferred_element_type=jnp.float32)
         + b_ref[...])
    zs = jnp.stack([z[:, :32], z[:, 32:]])
    z_ref[...] = zs
    row = i * BLK + lax.broadcasted_iota(jnp.int32, (1, BLK, 1), 1)
    mask = row < N
    zm = jnp.where(mask, zs, 0.0)

    @pl.when(i == 0)
    def _():
        parts_ref[...] = jnp.zeros_like(parts_ref)

    parts_ref[0] += jnp.sum(zm, axis=1)
    parts_ref[1] += jnp.sum(zm * zs, axis=1)


def _tc_bn_relu_body(z_ref, parts_ref, g_ref, bt_ref, h_ref):
    m = parts_ref[0] * (1.0 / N)
    v = parts_ref[1] * (1.0 / N) - m * m
    scale = g_ref[...] * lax.rsqrt(v + EPS)
    shift = bt_ref[...] - m * scale
    h_ref[...] = jnp.maximum(
        z_ref[...] * scale[:, None, :] + shift[:, None, :], 0.0)


def _tc3b_body(z_ref, parts_ref, g_ref, bt_ref, batch_ref, w_ref, bo_ref,
               out_ref, acc_ref):
    i = pl.program_id(0)
    m = parts_ref[0] * (1.0 / N)
    v = parts_ref[1] * (1.0 / N) - m * m
    scale = g_ref[...] * lax.rsqrt(v + EPS)
    shift = bt_ref[...] - m * scale
    h3 = z_ref[...] * scale[:, None, :] + shift[:, None, :]
    t = h3 * w_ref[...][:, None, :]
    sfull = jnp.sum(t, axis=0)
    s = jnp.sum(sfull, axis=1, keepdims=True)
    b = batch_ref[...]
    oh = (b == lax.broadcasted_iota(jnp.int32, (1, G), 1)).astype(jnp.float32)

    @pl.when(i == 0)
    def _():
        acc_ref[...] = jnp.zeros_like(acc_ref)

    acc_ref[0:1] += jnp.sum(oh * s, axis=0, keepdims=True)
    acc_ref[1:2] += jnp.sum(oh, axis=0, keepdims=True)

    @pl.when(i == NUM_BLK - 1)
    def _():
        ps = acc_ref[0:1]
        cnt = jnp.maximum(acc_ref[1:2], 1.0)
        out_ref[...] = ps / cnt + bo_ref[0, 0]


def _node_spec(rank3=False):
    if rank3:
        return pl.BlockSpec((2, BLK, 32), lambda i: (0, i, 0))
    return pl.BlockSpec((BLK, 1), lambda i: (i, 0))


def _full_spec(shape):
    nd = len(shape)
    return pl.BlockSpec(shape, lambda i: (0,) * nd)


def kernel(x, edge_index, batch, W_rel1, b_rel1, W_root1, gamma1, beta1,
           W_rel2, b_rel2, W_root2, gamma2, beta2, W_rel3, b_rel3, W_root3,
           gamma3, beta3, W_out, b_out):
    f32 = jnp.float32
    src = edge_index[0]
    dst = edge_index[1]
    pad_e = E_ROWS * 128 - E
    src2d = jnp.concatenate(
        [src, jnp.full((pad_e,), DUMMY, jnp.int32)]).reshape(E_ROWS, 128)
    dst2d = jnp.concatenate(
        [dst, jnp.full((pad_e,), DUMMY, jnp.int32)]).reshape(E_ROWS, 128)
    x_pad = jnp.pad(x, ((0, N_PAD - N), (0, 0)))
    batch2d = jnp.pad(batch, (0, N_PAD - N),
                      constant_values=G).reshape(N_PAD, 1)
    zeros1 = jnp.zeros((N_PAD, 1), f32)
    zeros32 = jnp.zeros((N_PAD, 32), f32)

    al1 = W_rel1[:, 0].reshape(2, 32)
    ber1 = W_root1[:, 0].reshape(2, 32)
    bb1 = b_rel1.reshape(2, 32)
    g1 = gamma1.reshape(2, 32)
    bt1 = beta1.reshape(2, 32)
    g2 = gamma2.reshape(2, 32)
    bt2 = beta2.reshape(2, 32)
    g3 = gamma3.reshape(2, 32)
    bt3 = beta3.reshape(2, 32)
    wr2 = W_rel2.T
    wo2 = W_root2.T
    br2 = b_rel2.reshape(1, 64)
    wr3 = W_rel3.T
    wo3 = W_root3.T
    br3 = b_rel3.reshape(1, 64)
    wout = W_out[0].reshape(2, 32)
    bo = b_out.reshape(1, 1)

    agg1p = _sc_agg_scalar(x_pad, src2d, dst2d, zeros1)

    a, parts1 = pl.pallas_call(
        _tc1a_body,
        grid=(NUM_BLK,),
        in_specs=[pl.BlockSpec((2, BLK, 1), lambda i: (0, i, 0)),
                  _node_spec()],
        out_specs=[_node_spec(), _full_spec((BLK, 8))],
        out_shape=[jax.ShapeDtypeStruct((N_PAD, 1), f32),
                   jax.ShapeDtypeStruct((BLK, 8), f32)],
    )(agg1p, x_pad)

    h1 = pl.pallas_call(
        _tc1b_body,
        grid=(NUM_BLK,),
        in_specs=[_node_spec(), _node_spec(), _full_spec((BLK, 8)),
                  _full_spec((2, 32)), _full_spec((2, 32)),
                  _full_spec((2, 32)), _full_spec((2, 32)),
                  _full_spec((2, 32))],
        out_specs=_node_spec(rank3=True),
        out_shape=jax.ShapeDtypeStruct((2, N_PAD, 32), f32),
    )(a, x_pad, parts1, al1, ber1, bb1, g1, bt1)

    agg2 = _sc_agg_h(h1, src2d, dst2d, zeros32)

    z2, parts2 = pl.pallas_call(
        _tc_layer_body,
        grid=(NUM_BLK,),
        in_specs=[_node_spec(rank3=True), _node_spec(rank3=True),
                  _full_spec((64, 64)), _full_spec((64, 64)),
                  _full_spec((1, 64))],
        out_specs=[_node_spec(rank3=True), _full_spec((2, 2, 32))],
        out_shape=[jax.ShapeDtypeStruct((2, N_PAD, 32), f32),
                   jax.ShapeDtypeStruct((2, 2, 32), f32)],
    )(agg2, h1, wr2, wo2, br2)

    h2 = pl.pallas_call(
        _tc_bn_relu_body,
        grid=(NUM_BLK,),
        in_specs=[_node_spec(rank3=True), _full_spec((2, 2, 32)),
                  _full_spec((2, 32)), _full_spec((2, 32))],
        out_specs=_node_spec(rank3=True),
        out_shape=jax.ShapeDtypeStruct((2, N_PAD, 32), f32),
    )(z2, parts2, g2, bt2)

    agg3 = _sc_agg_h(h2, src2d, dst2d, zeros32)

    z3, parts3 = pl.pallas_call(
        _tc_layer_body,
        grid=(NUM_BLK,),
        in_specs=[_node_spec(rank3=True), _node_spec(rank3=True),
                  _full_spec((64, 64)), _full_spec((64, 64)),
                  _full_spec((1, 64))],
        out_specs=[_node_spec(rank3=True), _full_spec((2, 2, 32))],
        out_shape=[jax.ShapeDtypeStruct((2, N_PAD, 32), f32),
                   jax.ShapeDtypeStruct((2, 2, 32), f32)],
    )(agg3, h2, wr3, wo3, br3)

    out = pl.pallas_call(
        _tc3b_body,
        grid=(NUM_BLK,),
        in_specs=[_node_spec(rank3=True), _full_spec((2, 2, 32)),
                  _full_spec((2, 32)), _full_spec((2, 32)),
                  _node_spec(), _full_spec((2, 32)), _full_spec((1, 1))],
        out_specs=_full_spec((1, G)),
        out_shape=jax.ShapeDtypeStruct((1, G), f32),
        scratch_shapes=[pltpu.VMEM((2, G), f32)],
    )(z3, parts3, g3, bt3, batch2d, wout, bo)

    return out.reshape(G, 1)


# speedup vs baseline: 6.9785x; 6.9785x over previous
import functools

import jax
import jax.numpy as jnp
from jax import lax
from jax.experimental import pallas as pl
from jax.experimental.pallas import tpu as pltpu
from jax.experimental.pallas import tpu_sc as plsc

N = 50000
E = 800000
H = 64
G = 64
EPS = 1e-5

N_PAD = 50176
DUMMY = N
E_ROWS = 6400
BLK = 1024
NUM_BLK = N_PAD // BLK
SUBSL = N_PAD // 16
SUP = 8
SUPH = 4


def _sc_agg_scalar(x_pad, src2d, dst2d, zeros1):
    mesh = plsc.VectorSubcoreMesh(core_axis_name="c", subcore_axis_name="s", num_cores=2, num_subcores=16)

    @functools.partial(
        pl.kernel,
        out_type=jax.ShapeDtypeStruct((2, N_PAD, 16), jnp.float32),
        mesh=mesh,
        scratch_types=[
            pltpu.VMEM_SHARED((N_PAD, 16), jnp.float32),
            pltpu.VMEM((SUP, 128), jnp.int32),
            pltpu.VMEM((SUP, 128), jnp.int32),
            pltpu.VMEM((SUP * 128, 16), jnp.float32),
            pltpu.SemaphoreType.DMA,
        ],
        compiler_params=pltpu.CompilerParams(use_tc_tiling_on_sc=False),
    )
    def k(x_hbm, src_hbm, dst_hbm, zz_hbm, out_hbm, acc, sidx, didx, rows, sem):
        c = lax.axis_index("c")
        s = lax.axis_index("s")
        w0 = s * SUBSL
        pltpu.sync_copy(zz_hbm.at[pl.ds(w0, SUBSL)], acc.at[pl.ds(w0, SUBSL)])
        plsc.subcore_barrier()

        def body(i, carry):
            r0 = c * 3200 + s * 200 + i * SUP
            pltpu.sync_copy(src_hbm.at[pl.ds(r0, SUP)], sidx)
            pltpu.sync_copy(dst_hbm.at[pl.ds(r0, SUP)], didx)
            cps = [
                pltpu.async_copy(x_hbm.at[sidx.at[j]],
                                 rows.at[pl.ds(j * 128, 128)], sem)
                for j in range(SUP)
            ]
            for cp in cps:
                cp.wait()
            for j in range(SUP):
                pltpu.sync_copy(rows.at[pl.ds(j * 128, 128)],
                                acc.at[didx.at[j]], add=True)
            return carry

        lax.fori_loop(0, 25, body, 0)
        plsc.subcore_barrier()

        @pl.when(c == 0)
        def _():
            pltpu.sync_copy(acc.at[pl.ds(w0, SUBSL)],
                            out_hbm.at[0].at[pl.ds(w0, SUBSL)])

        @pl.when(c == 1)
        def _():
            pltpu.sync_copy(acc.at[pl.ds(w0, SUBSL)],
                            out_hbm.at[1].at[pl.ds(w0, SUBSL)])

    return k(x_pad, src2d, dst2d, zeros1)


def _sc_agg_h(h_split, src2d, dst2d, zeros32):
    mesh = plsc.VectorSubcoreMesh(core_axis_name="c", subcore_axis_name="s", num_cores=2, num_subcores=16)

    @functools.partial(
        pl.kernel,
        out_type=jax.ShapeDtypeStruct((2, N_PAD, 32), jnp.float32),
        mesh=mesh,
        scratch_types=[
            pltpu.VMEM_SHARED((N_PAD, 32), jnp.float32),
            pltpu.VMEM((SUPH, 128), jnp.int32),
            pltpu.VMEM((SUPH, 128), jnp.int32),
            pltpu.VMEM((SUPH * 128, 32), jnp.float32),
            pltpu.SemaphoreType.DMA,
        ],
        compiler_params=pltpu.CompilerParams(use_tc_tiling_on_sc=False),
    )
    def k(h_hbm, src_hbm, dst_hbm, zz_hbm, out_hbm, acc, sidx, didx, rows, sem):
        c = lax.axis_index("c")
        s = lax.axis_index("s")
        w0 = s * SUBSL
        pltpu.sync_copy(zz_hbm.at[pl.ds(w0, SUBSL)], acc.at[pl.ds(w0, SUBSL)])
        plsc.subcore_barrier()

        def make_body(h_view):
            def body(i, carry):
                r0 = s * 400 + i * SUPH
                pltpu.sync_copy(src_hbm.at[pl.ds(r0, SUPH)], sidx)
                pltpu.sync_copy(dst_hbm.at[pl.ds(r0, SUPH)], didx)
                cps = [
                    pltpu.async_copy(h_view.at[sidx.at[j]],
                                     rows.at[pl.ds(j * 128, 128)], sem)
                    for j in range(SUPH)
                ]
                for cp in cps:
                    cp.wait()
                for j in range(SUPH):
                    pltpu.sync_copy(rows.at[pl.ds(j * 128, 128)],
                                    acc.at[didx.at[j]], add=True)
                return carry
            return body

        @pl.when(c == 0)
        def _():
            lax.fori_loop(0, 100, make_body(h_hbm.at[0]), 0)

        @pl.when(c == 1)
        def _():
            lax.fori_loop(0, 100, make_body(h_hbm.at[1]), 0)

        plsc.subcore_barrier()

        @pl.when(c == 0)
        def _():
            pltpu.sync_copy(acc.at[pl.ds(w0, SUBSL)],
                            out_hbm.at[0].at[pl.ds(w0, SUBSL)])

        @pl.when(c == 1)
        def _():
            pltpu.sync_copy(acc.at[pl.ds(w0, SUBSL)],
                            out_hbm.at[1].at[pl.ds(w0, SUBSL)])

    return k(h_split, src2d, dst2d, zeros32)


def _tc1a_body(aggp_ref, x_ref, a_ref, parts_ref):
    i = pl.program_id(0)
    a = aggp_ref[0] + aggp_ref[1]
    a_ref[...] = a
    x = x_ref[...]
    row = i * BLK + lax.broadcasted_iota(jnp.int32, (BLK, 1), 0)
    mask = row < N
    am = jnp.where(mask, a, 0.0)
    xm = jnp.where(mask, x, 0.0)

    @pl.when(i == 0)
    def _():
        parts_ref[...] = jnp.zeros_like(parts_ref)

    parts_ref[:, 0:1] += am
    parts_ref[:, 1:2] += am * am
    parts_ref[:, 2:3] += xm
    parts_ref[:, 3:4] += xm * xm
    parts_ref[:, 4:5] += am * xm


def _tc1b_body(a_ref, x_ref, parts_ref, al_ref, ber_ref, bb_ref, g_ref,
               bt_ref, h_ref):
    p = parts_ref[...]
    ninv = 1.0 / N
    ma = jnp.sum(p[:, 0]) * ninv
    saa = jnp.sum(p[:, 1]) * ninv
    mx = jnp.sum(p[:, 2]) * ninv
    sxx = jnp.sum(p[:, 3]) * ninv
    sax = jnp.sum(p[:, 4]) * ninv
    va = saa - ma * ma
    vx = sxx - mx * mx
    cax = sax - ma * mx
    al = al_ref[...]
    ber = ber_ref[...]
    bb = bb_ref[...]
    m = ma * al + mx * ber + bb
    v = al * al * va + 2.0 * al * ber * cax + ber * ber * vx
    scale = g_ref[...] * lax.rsqrt(v + EPS)
    A = al * scale
    B = ber * scale
    C = (bb - m) * scale + bt_ref[...]
    a = a_ref[...]
    x = x_ref[...]
    h = a[None, :, :] * A[:, None, :] + x[None, :, :] * B[:, None, :] \
        + C[:, None, :]
    h_ref[...] = jnp.maximum(h, 0.0)


def _tc_layer_body(agg_ref, h_ref, wr_ref, wo_ref, b_ref, z_ref, parts_ref):
    i = pl.program_id(0)
    a = jnp.concatenate([agg_ref[0], agg_ref[1]], axis=1)
    h = jnp.concatenate([h_ref[0], h_ref[1]], axis=1)
    z = (jnp.dot(a, wr_ref[...], preferred_element_type=jnp.float32,
             precision=lax.Precision.HIGHEST)
         + jnp.dot(h, wo_ref[...], preferred_element_type=jnp.float32,
               precision=lax.Precision.HIGHEST)
         + b_ref[...])
    zs = jnp.stack([z[:, :32], z[:, 32:]])
    z_ref[...] = zs
    row = i * BLK + lax.broadcasted_iota(jnp.int32, (1, BLK, 1), 1)
    mask = row < N
    zm = jnp.where(mask, zs, 0.0)

    @pl.when(i == 0)
    def _():
        parts_ref[...] = jnp.zeros_like(parts_ref)

    parts_ref[0] += jnp.sum(zm, axis=1)
    parts_ref[1] += jnp.sum(zm * zs, axis=1)


def _tc_bn_relu_body(z_ref, parts_ref, g_ref, bt_ref, h_ref):
    m = parts_ref[0] * (1.0 / N)
    v = parts_ref[1] * (1.0 / N) - m * m
    scale = g_ref[...] * lax.rsqrt(v + EPS)
    shift = bt_ref[...] - m * scale
    h_ref[...] = jnp.maximum(
        z_ref[...] * scale[:, None, :] + shift[:, None, :], 0.0)


def _tc3b_body(z_ref, parts_ref, g_ref, bt_ref, batch_ref, w_ref, bo_ref,
               out_ref, acc_ref):
    i = pl.program_id(0)
    m = parts_ref[0] * (1.0 / N)
    v = parts_ref[1] * (1.0 / N) - m * m
    scale = g_ref[...] * lax.rsqrt(v + EPS)
    shift = bt_ref[...] - m * scale
    h3 = z_ref[...] * scale[:, None, :] + shift[:, None, :]
    t = h3 * w_ref[...][:, None, :]
    sfull = jnp.sum(t, axis=0)
    s = jnp.sum(sfull, axis=1, keepdims=True)
    b = batch_ref[...]
    oh = (b == lax.broadcasted_iota(jnp.int32, (1, G), 1)).astype(jnp.float32)

    @pl.when(i == 0)
    def _():
        acc_ref[...] = jnp.zeros_like(acc_ref)

    acc_ref[0:1] += jnp.sum(oh * s, axis=0, keepdims=True)
    acc_ref[1:2] += jnp.sum(oh, axis=0, keepdims=True)

    @pl.when(i == NUM_BLK - 1)
    def _():
        ps = acc_ref[0:1]
        cnt = jnp.maximum(acc_ref[1:2], 1.0)
        out_ref[...] = ps / cnt + bo_ref[0, 0]


def _node_spec(rank3=False):
    if rank3:
        return pl.BlockSpec((2, BLK, 32), lambda i: (0, i, 0))
    return pl.BlockSpec((BLK, 1), lambda i: (i, 0))


def _full_spec(shape):
    nd = len(shape)
    return pl.BlockSpec(shape, lambda i: (0,) * nd)


def kernel(x, edge_index, batch, W_rel1, b_rel1, W_root1, gamma1, beta1,
           W_rel2, b_rel2, W_root2, gamma2, beta2, W_rel3, b_rel3, W_root3,
           gamma3, beta3, W_out, b_out):
    f32 = jnp.float32
    src = edge_index[0]
    dst = edge_index[1]
    pad_e = E_ROWS * 128 - E
    src2d = jnp.concatenate(
        [src, jnp.full((pad_e,), DUMMY, jnp.int32)]).reshape(E_ROWS, 128)
    dst2d = jnp.concatenate(
        [dst, jnp.full((pad_e,), DUMMY, jnp.int32)]).reshape(E_ROWS, 128)
    x_pad = jnp.pad(x, ((0, N_PAD - N), (0, 0)))
    x_pad16 = jnp.pad(x, ((0, N_PAD - N), (0, 15)))
    batch2d = jnp.pad(batch, (0, N_PAD - N),
                      constant_values=G).reshape(N_PAD, 1)
    zeros16 = jnp.zeros((N_PAD, 16), f32)
    zeros32 = jnp.zeros((N_PAD, 32), f32)

    al1 = W_rel1[:, 0].reshape(2, 32)
    ber1 = W_root1[:, 0].reshape(2, 32)
    bb1 = b_rel1.reshape(2, 32)
    g1 = gamma1.reshape(2, 32)
    bt1 = beta1.reshape(2, 32)
    g2 = gamma2.reshape(2, 32)
    bt2 = beta2.reshape(2, 32)
    g3 = gamma3.reshape(2, 32)
    bt3 = beta3.reshape(2, 32)
    wr2 = W_rel2.T
    wo2 = W_root2.T
    br2 = b_rel2.reshape(1, 64)
    wr3 = W_rel3.T
    wo3 = W_root3.T
    br3 = b_rel3.reshape(1, 64)
    wout = W_out[0].reshape(2, 32)
    bo = b_out.reshape(1, 1)

    agg1p = _sc_agg_scalar(x_pad16, src2d, dst2d, zeros16)[:, :, 0:1]

    a, parts1 = pl.pallas_call(
        _tc1a_body,
        grid=(NUM_BLK,),
        in_specs=[pl.BlockSpec((2, BLK, 1), lambda i: (0, i, 0)),
                  _node_spec()],
        out_specs=[_node_spec(), _full_spec((BLK, 8))],
        out_shape=[jax.ShapeDtypeStruct((N_PAD, 1), f32),
                   jax.ShapeDtypeStruct((BLK, 8), f32)],
    )(agg1p, x_pad)

    h1 = pl.pallas_call(
        _tc1b_body,
        grid=(NUM_BLK,),
        in_specs=[_node_spec(), _node_spec(), _full_spec((BLK, 8)),
                  _full_spec((2, 32)), _full_spec((2, 32)),
                  _full_spec((2, 32)), _full_spec((2, 32)),
                  _full_spec((2, 32))],
        out_specs=_node_spec(rank3=True),
        out_shape=jax.ShapeDtypeStruct((2, N_PAD, 32), f32),
    )(a, x_pad, parts1, al1, ber1, bb1, g1, bt1)

    agg2 = _sc_agg_h(h1, src2d, dst2d, zeros32)

    z2, parts2 = pl.pallas_call(
        _tc_layer_body,
        grid=(NUM_BLK,),
        in_specs=[_node_spec(rank3=True), _node_spec(rank3=True),
                  _full_spec((64, 64)), _full_spec((64, 64)),
                  _full_spec((1, 64))],
        out_specs=[_node_spec(rank3=True), _full_spec((2, 2, 32))],
        out_shape=[jax.ShapeDtypeStruct((2, N_PAD, 32), f32),
                   jax.ShapeDtypeStruct((2, 2, 32), f32)],
    )(agg2, h1, wr2, wo2, br2)

    h2 = pl.pallas_call(
        _tc_bn_relu_body,
        grid=(NUM_BLK,),
        in_specs=[_node_spec(rank3=True), _full_spec((2, 2, 32)),
                  _full_spec((2, 32)), _full_spec((2, 32))],
        out_specs=_node_spec(rank3=True),
        out_shape=jax.ShapeDtypeStruct((2, N_PAD, 32), f32),
    )(z2, parts2, g2, bt2)

    agg3 = _sc_agg_h(h2, src2d, dst2d, zeros32)

    z3, parts3 = pl.pallas_call(
        _tc_layer_body,
        grid=(NUM_BLK,),
        in_specs=[_node_spec(rank3=True), _node_spec(rank3=True),
                  _full_spec((64, 64)), _full_spec((64, 64)),
                  _full_spec((1, 64))],
        out_specs=[_node_spec(rank3=True), _full_spec((2, 2, 32))],
        out_shape=[jax.ShapeDtypeStruct((2, N_PAD, 32), f32),
                   jax.ShapeDtypeStruct((2, 2, 32), f32)],
    )(agg3, h2, wr3, wo3, br3)

    out = pl.pallas_call(
        _tc3b_body,
        grid=(NUM_BLK,),
        in_specs=[_node_spec(rank3=True), _full_spec((2, 2, 32)),
                  _full_spec((2, 32)), _full_spec((2, 32)),
                  _node_spec(), _full_spec((2, 32)), _full_spec((1, 1))],
        out_specs=_full_spec((1, G)),
        out_shape=jax.ShapeDtypeStruct((1, G), f32),
        scratch_shapes=[pltpu.VMEM((2, G), f32)],
    )(z3, parts3, g3, bt3, batch2d, wout, bo)

    return out.reshape(G, 1)


# speedup vs baseline: 7.7150x; 1.1055x over previous
import functools

import jax
import jax.numpy as jnp
from jax import lax
from jax.experimental import pallas as pl
from jax.experimental.pallas import tpu as pltpu
from jax.experimental.pallas import tpu_sc as plsc

N = 50000
E = 800000
H = 64
G = 64
EPS = 1e-5

N_PAD = 50176
DUMMY = N
E_ROWS = 6400
BLK = 1024
NUM_BLK = N_PAD // BLK
SUBSL = N_PAD // 16
SUP = 8
SUPH = 4


TBLK = 40
RING = 4


def _sc_edge_loop(h_view, e_hbm, acc, eidx, rows, gsems, ssems, base):
    pltpu.sync_copy(e_hbm.at[pl.ds(base, 2 * TBLK)], eidx)
    for g in range(TBLK // RING):
        gd = []
        for b in range(RING):
            j = g * RING + b
            gd.append(pltpu.async_copy(h_view.at[eidx.at[j]], rows[b],
                                       gsems[b]))
        for cp in gd:
            cp.wait()
        for b in range(RING):
            j = g * RING + b
            pltpu.sync_copy(rows[b], acc.at[eidx.at[TBLK + j]], add=True)


def _sc_agg_scalar(x_pad, e2, zeros16):
    mesh = plsc.VectorSubcoreMesh(core_axis_name="c", subcore_axis_name="s",
                                  num_cores=2, num_subcores=16)

    @functools.partial(
        pl.kernel,
        out_type=jax.ShapeDtypeStruct((2, N_PAD, 16), jnp.float32),
        mesh=mesh,
        scratch_types=[
            pltpu.VMEM_SHARED((N_PAD, 16), jnp.float32),
            pltpu.VMEM((2 * TBLK, 128), jnp.int32),
        ] + [pltpu.VMEM((128, 16), jnp.float32)] * RING
          + [pltpu.SemaphoreType.DMA] * (2 * RING),
        compiler_params=pltpu.CompilerParams(use_tc_tiling_on_sc=False),
    )
    def k(x_hbm, e_hbm, zz_hbm, out_hbm, acc, eidx, r0, r1, r2, r3,
          g0, g1, g2, g3, s0, s1, s2, s3):
        rows = [r0, r1, r2, r3]
        gsems = [g0, g1, g2, g3]
        ssems = [s0, s1, s2, s3]
        c = lax.axis_index("c")
        s = lax.axis_index("s")
        w0 = s * SUBSL
        pltpu.sync_copy(zz_hbm.at[pl.ds(w0, SUBSL)], acc.at[pl.ds(w0, SUBSL)])
        plsc.subcore_barrier()

        def body(kk, carry):
            base = ((c * 16 + s) * 5 + kk) * (2 * TBLK)
            _sc_edge_loop(x_hbm, e_hbm, acc, eidx, rows, gsems, ssems, base)
            return carry

        lax.fori_loop(0, 5, body, 0)
        plsc.subcore_barrier()

        @pl.when(c == 0)
        def _():
            pltpu.sync_copy(acc.at[pl.ds(w0, SUBSL)],
                            out_hbm.at[0].at[pl.ds(w0, SUBSL)])

        @pl.when(c == 1)
        def _():
            pltpu.sync_copy(acc.at[pl.ds(w0, SUBSL)],
                            out_hbm.at[1].at[pl.ds(w0, SUBSL)])

    return k(x_pad, e2, zeros16)


def _sc_agg_h(h_split, e2, zeros32):
    mesh = plsc.VectorSubcoreMesh(core_axis_name="c", subcore_axis_name="s",
                                  num_cores=2, num_subcores=16)

    @functools.partial(
        pl.kernel,
        out_type=jax.ShapeDtypeStruct((2, N_PAD, 32), jnp.float32),
        mesh=mesh,
        scratch_types=[
            pltpu.VMEM_SHARED((N_PAD, 32), jnp.float32),
            pltpu.VMEM((2 * TBLK, 128), jnp.int32),
        ] + [pltpu.VMEM((128, 32), jnp.float32)] * RING
          + [pltpu.SemaphoreType.DMA] * (2 * RING),
        compiler_params=pltpu.CompilerParams(use_tc_tiling_on_sc=False),
    )
    def k(h_hbm, e_hbm, zz_hbm, out_hbm, acc, eidx, r0, r1, r2, r3,
          g0, g1, g2, g3, s0, s1, s2, s3):
        rows = [r0, r1, r2, r3]
        gsems = [g0, g1, g2, g3]
        ssems = [s0, s1, s2, s3]
        c = lax.axis_index("c")
        s = lax.axis_index("s")
        w0 = s * SUBSL
        pltpu.sync_copy(zz_hbm.at[pl.ds(w0, SUBSL)], acc.at[pl.ds(w0, SUBSL)])
        plsc.subcore_barrier()

        def make_body(h_view):
            def body(kk, carry):
                base = (s * 10 + kk) * (2 * TBLK)
                _sc_edge_loop(h_view, e_hbm, acc, eidx, rows, gsems, ssems,
                              base)
                return carry
            return body

        @pl.when(c == 0)
        def _():
            lax.fori_loop(0, 10, make_body(h_hbm.at[0]), 0)

        @pl.when(c == 1)
        def _():
            lax.fori_loop(0, 10, make_body(h_hbm.at[1]), 0)

        plsc.subcore_barrier()

        @pl.when(c == 0)
        def _():
            pltpu.sync_copy(acc.at[pl.ds(w0, SUBSL)],
                            out_hbm.at[0].at[pl.ds(w0, SUBSL)])

        @pl.when(c == 1)
        def _():
            pltpu.sync_copy(acc.at[pl.ds(w0, SUBSL)],
                            out_hbm.at[1].at[pl.ds(w0, SUBSL)])

    return k(h_split, e2, zeros32)


def _tc1a_body(aggp_ref, x_ref, a_ref, parts_ref):
    i = pl.program_id(0)
    a = aggp_ref[0] + aggp_ref[1]
    a_ref[...] = a
    x = x_ref[...]
    row = i * BLK + lax.broadcasted_iota(jnp.int32, (BLK, 1), 0)
    mask = row < N
    am = jnp.where(mask, a, 0.0)
    xm = jnp.where(mask, x, 0.0)

    @pl.when(i == 0)
    def _():
        parts_ref[...] = jnp.zeros_like(parts_ref)

    parts_ref[:, 0:1] += am
    parts_ref[:, 1:2] += am * am
    parts_ref[:, 2:3] += xm
    parts_ref[:, 3:4] += xm * xm
    parts_ref[:, 4:5] += am * xm


def _tc1b_body(a_ref, x_ref, parts_ref, al_ref, ber_ref, bb_ref, g_ref,
               bt_ref, h_ref):
    p = parts_ref[...]
    ninv = 1.0 / N
    ma = jnp.sum(p[:, 0]) * ninv
    saa = jnp.sum(p[:, 1]) * ninv
    mx = jnp.sum(p[:, 2]) * ninv
    sxx = jnp.sum(p[:, 3]) * ninv
    sax = jnp.sum(p[:, 4]) * ninv
    va = saa - ma * ma
    vx = sxx - mx * mx
    cax = sax - ma * mx
    al = al_ref[...]
    ber = ber_ref[...]
    bb = bb_ref[...]
    m = ma * al + mx * ber + bb
    v = al * al * va + 2.0 * al * ber * cax + ber * ber * vx
    scale = g_ref[...] * lax.rsqrt(v + EPS)
    A = al * scale
    B = ber * scale
    C = (bb - m) * scale + bt_ref[...]
    a = a_ref[...]
    x = x_ref[...]
    h = a[None, :, :] * A[:, None, :] + x[None, :, :] * B[:, None, :] \
        + C[:, None, :]
    h_ref[...] = jnp.maximum(h, 0.0)


def _tc_layer_body(agg_ref, h_ref, wr_ref, wo_ref, b_ref, z_ref, parts_ref):
    i = pl.program_id(0)
    a = jnp.concatenate([agg_ref[0], agg_ref[1]], axis=1)
    h = jnp.concatenate([h_ref[0], h_ref[1]], axis=1)
    z = (jnp.dot(a, wr_ref[...], preferred_element_type=jnp.float32,
             precision=lax.Precision.HIGHEST)
         + jnp.dot(h, wo_ref[...], preferred_element_type=jnp.float32,
               precision=lax.Precision.HIGHEST)
         + b_ref[...])
    zs = jnp.stack([z[:, :32], z[:, 32:]])
    z_ref[...] = zs
    row = i * BLK + lax.broadcasted_iota(jnp.int32, (1, BLK, 1), 1)
    mask = row < N
    zm = jnp.where(mask, zs, 0.0)

    @pl.when(i == 0)
    def _():
        parts_ref[...] = jnp.zeros_like(parts_ref)

    parts_ref[0] += jnp.sum(zm, axis=1)
    parts_ref[1] += jnp.sum(zm * zs, axis=1)


def _tc_bn_relu_body(z_ref, parts_ref, g_ref, bt_ref, h_ref):
    m = parts_ref[0] * (1.0 / N)
    v = parts_ref[1] * (1.0 / N) - m * m
    scale = g_ref[...] * lax.rsqrt(v + EPS)
    shift = bt_ref[...] - m * scale
    h_ref[...] = jnp.maximum(
        z_ref[...] * scale[:, None, :] + shift[:, None, :], 0.0)


def _tc3b_body(z_ref, parts_ref, g_ref, bt_ref, batch_ref, w_ref, bo_ref,
               out_ref, acc_ref):
    i = pl.program_id(0)
    m = parts_ref[0] * (1.0 / N)
    v = parts_ref[1] * (1.0 / N) - m * m
    scale = g_ref[...] * lax.rsqrt(v + EPS)
    shift = bt_ref[...] - m * scale
    h3 = z_ref[...] * scale[:, None, :] + shift[:, None, :]
    t = h3 * w_ref[...][:, None, :]
    sfull = jnp.sum(t, axis=0)
    s = jnp.sum(sfull, axis=1, keepdims=True)
    b = batch_ref[...]
    oh = (b == lax.broadcasted_iota(jnp.int32, (1, G), 1)).astype(jnp.float32)

    @pl.when(i == 0)
    def _():
        acc_ref[...] = jnp.zeros_like(acc_ref)

    acc_ref[0:1] += jnp.sum(oh * s, axis=0, keepdims=True)
    acc_ref[1:2] += jnp.sum(oh, axis=0, keepdims=True)

    @pl.when(i == NUM_BLK - 1)
    def _():
        ps = acc_ref[0:1]
        cnt = jnp.maximum(acc_ref[1:2], 1.0)
        out_ref[...] = ps / cnt + bo_ref[0, 0]


def _node_spec(rank3=False):
    if rank3:
        return pl.BlockSpec((2, BLK, 32), lambda i: (0, i, 0))
    return pl.BlockSpec((BLK, 1), lambda i: (i, 0))


def _full_spec(shape):
    nd = len(shape)
    return pl.BlockSpec(shape, lambda i: (0,) * nd)


def kernel(x, edge_index, batch, W_rel1, b_rel1, W_root1, gamma1, beta1,
           W_rel2, b_rel2, W_root2, gamma2, beta2, W_rel3, b_rel3, W_root3,
           gamma3, beta3, W_out, b_out):
    f32 = jnp.float32
    src = edge_index[0]
    dst = edge_index[1]
    pad_e = E_ROWS * 128 - E
    src2d = jnp.concatenate(
        [src, jnp.full((pad_e,), DUMMY, jnp.int32)]).reshape(E_ROWS, 128)
    dst2d = jnp.concatenate(
        [dst, jnp.full((pad_e,), DUMMY, jnp.int32)]).reshape(E_ROWS, 128)
    sh = src2d.reshape(16, 10, TBLK, 128)
    dh = dst2d.reshape(16, 10, TBLK, 128)
    e2h = jnp.concatenate([sh, dh], axis=2).reshape(-1, 128)
    s1 = src2d.reshape(2, 16, 5, TBLK, 128)
    d1 = dst2d.reshape(2, 16, 5, TBLK, 128)
    e21 = jnp.concatenate([s1, d1], axis=3).reshape(-1, 128)
    x_pad = jnp.pad(x, ((0, N_PAD - N), (0, 0)))
    x_pad16 = jnp.pad(x, ((0, N_PAD - N), (0, 15)))
    batch2d = jnp.pad(batch, (0, N_PAD - N),
                      constant_values=G).reshape(N_PAD, 1)
    zeros16 = jnp.zeros((N_PAD, 16), f32)
    zeros32 = jnp.zeros((N_PAD, 32), f32)

    al1 = W_rel1[:, 0].reshape(2, 32)
    ber1 = W_root1[:, 0].reshape(2, 32)
    bb1 = b_rel1.reshape(2, 32)
    g1 = gamma1.reshape(2, 32)
    bt1 = beta1.reshape(2, 32)
    g2 = gamma2.reshape(2, 32)
    bt2 = beta2.reshape(2, 32)
    g3 = gamma3.reshape(2, 32)
    bt3 = beta3.reshape(2, 32)
    wr2 = W_rel2.T
    wo2 = W_root2.T
    br2 = b_rel2.reshape(1, 64)
    wr3 = W_rel3.T
    wo3 = W_root3.T
    br3 = b_rel3.reshape(1, 64)
    wout = W_out[0].reshape(2, 32)
    bo = b_out.reshape(1, 1)

    agg1p = _sc_agg_scalar(x_pad16, e21, zeros16)[:, :, 0:1]

    a, parts1 = pl.pallas_call(
        _tc1a_body,
        grid=(NUM_BLK,),
        in_specs=[pl.BlockSpec((2, BLK, 1), lambda i: (0, i, 0)),
                  _node_spec()],
        out_specs=[_node_spec(), _full_spec((BLK, 8))],
        out_shape=[jax.ShapeDtypeStruct((N_PAD, 1), f32),
                   jax.ShapeDtypeStruct((BLK, 8), f32)],
    )(agg1p, x_pad)

    h1 = pl.pallas_call(
        _tc1b_body,
        grid=(NUM_BLK,),
        in_specs=[_node_spec(), _node_spec(), _full_spec((BLK, 8)),
                  _full_spec((2, 32)), _full_spec((2, 32)),
                  _full_spec((2, 32)), _full_spec((2, 32)),
                  _full_spec((2, 32))],
        out_specs=_node_spec(rank3=True),
        out_shape=jax.ShapeDtypeStruct((2, N_PAD, 32), f32),
    )(a, x_pad, parts1, al1, ber1, bb1, g1, bt1)

    agg2 = _sc_agg_h(h1, e2h, zeros32)

    z2, parts2 = pl.pallas_call(
        _tc_layer_body,
        grid=(NUM_BLK,),
        in_specs=[_node_spec(rank3=True), _node_spec(rank3=True),
                  _full_spec((64, 64)), _full_spec((64, 64)),
                  _full_spec((1, 64))],
        out_specs=[_node_spec(rank3=True), _full_spec((2, 2, 32))],
        out_shape=[jax.ShapeDtypeStruct((2, N_PAD, 32), f32),
                   jax.ShapeDtypeStruct((2, 2, 32), f32)],
    )(agg2, h1, wr2, wo2, br2)

    h2 = pl.pallas_call(
        _tc_bn_relu_body,
        grid=(NUM_BLK,),
        in_specs=[_node_spec(rank3=True), _full_spec((2, 2, 32)),
                  _full_spec((2, 32)), _full_spec((2, 32))],
        out_specs=_node_spec(rank3=True),
        out_shape=jax.ShapeDtypeStruct((2, N_PAD, 32), f32),
    )(z2, parts2, g2, bt2)

    agg3 = _sc_agg_h(h2, e2h, zeros32)

    z3, parts3 = pl.pallas_call(
        _tc_layer_body,
        grid=(NUM_BLK,),
        in_specs=[_node_spec(rank3=True), _node_spec(rank3=True),
                  _full_spec((64, 64)), _full_spec((64, 64)),
                  _full_spec((1, 64))],
        out_specs=[_node_spec(rank3=True), _full_spec((2, 2, 32))],
        out_shape=[jax.ShapeDtypeStruct((2, N_PAD, 32), f32),
                   jax.ShapeDtypeStruct((2, 2, 32), f32)],
    )(agg3, h2, wr3, wo3, br3)

    out = pl.pallas_call(
        _tc3b_body,
        grid=(NUM_BLK,),
        in_specs=[_node_spec(rank3=True), _full_spec((2, 2, 32)),
                  _full_spec((2, 32)), _full_spec((2, 32)),
                  _node_spec(), _full_spec((2, 32)), _full_spec((1, 1))],
        out_specs=_full_spec((1, G)),
        out_shape=jax.ShapeDtypeStruct((1, G), f32),
        scratch_shapes=[pltpu.VMEM((2, G), f32)],
    )(z3, parts3, g3, bt3, batch2d, wout, bo)

    return out.reshape(G, 1)


# speedup vs baseline: 7.7206x; 1.0007x over previous
import functools

import jax
import jax.numpy as jnp
from jax import lax
from jax.experimental import pallas as pl
from jax.experimental.pallas import tpu as pltpu
from jax.experimental.pallas import tpu_sc as plsc

N = 50000
E = 800000
H = 64
G = 64
EPS = 1e-5

N_PAD = 50176
DUMMY = N
E_ROWS = 6400
BLK = 1024
NUM_BLK = N_PAD // BLK
SUBSL = N_PAD // 16
SUP = 8
SUPH = 4


TBLK = 40
RING = 4


def _sc_edge_loop(h_view, e_hbm, acc, eidx, rows, gsems, base):
    pltpu.sync_copy(e_hbm.at[pl.ds(base, 2 * TBLK)], eidx)
    for g in range(TBLK // RING):
        gd = []
        for b in range(RING):
            j = g * RING + b
            gd.append(pltpu.async_copy(h_view.at[eidx.at[j]], rows[b],
                                       gsems[b]))
        for cp in gd:
            cp.wait()
        for b in range(RING):
            j = g * RING + b
            pltpu.sync_copy(rows[b], acc.at[eidx.at[TBLK + j]], add=True)


def _sc_agg_scalar(x_pad, e2, zeros16):
    mesh = plsc.VectorSubcoreMesh(core_axis_name="c", subcore_axis_name="s",
                                  num_cores=2, num_subcores=16)

    @functools.partial(
        pl.kernel,
        out_type=jax.ShapeDtypeStruct((2, N_PAD, 16), jnp.float32),
        mesh=mesh,
        scratch_types=[
            pltpu.VMEM_SHARED((N_PAD, 16), jnp.float32),
            pltpu.VMEM((2 * TBLK, 128), jnp.int32),
        ] + [pltpu.VMEM((128, 16), jnp.float32)] * RING
          + [pltpu.SemaphoreType.DMA] * RING,
        compiler_params=pltpu.CompilerParams(use_tc_tiling_on_sc=False),
    )
    def k(x_hbm, e_hbm, zz_hbm, out_hbm, acc, eidx, r0, r1, r2, r3,
          g0, g1, g2, g3):
        rows = [r0, r1, r2, r3]
        gsems = [g0, g1, g2, g3]
        c = lax.axis_index("c")
        s = lax.axis_index("s")
        w0 = s * SUBSL
        pltpu.sync_copy(zz_hbm.at[pl.ds(w0, SUBSL)], acc.at[pl.ds(w0, SUBSL)])
        plsc.subcore_barrier()

        def body(kk, carry):
            base = ((c * 16 + s) * 5 + kk) * (2 * TBLK)
            _sc_edge_loop(x_hbm, e_hbm, acc, eidx, rows, gsems, base)
            return carry

        lax.fori_loop(0, 5, body, 0)
        plsc.subcore_barrier()

        @pl.when(c == 0)
        def _():
            pltpu.sync_copy(acc.at[pl.ds(w0, SUBSL)],
                            out_hbm.at[0].at[pl.ds(w0, SUBSL)])

        @pl.when(c == 1)
        def _():
            pltpu.sync_copy(acc.at[pl.ds(w0, SUBSL)],
                            out_hbm.at[1].at[pl.ds(w0, SUBSL)])

    return k(x_pad, e2, zeros16)


def _sc_agg_h(h_split, e2, zeros32):
    mesh = plsc.VectorSubcoreMesh(core_axis_name="c", subcore_axis_name="s",
                                  num_cores=2, num_subcores=16)

    @functools.partial(
        pl.kernel,
        out_type=jax.ShapeDtypeStruct((2, N_PAD, 32), jnp.float32),
        mesh=mesh,
        scratch_types=[
            pltpu.VMEM_SHARED((N_PAD, 32), jnp.float32),
            pltpu.VMEM((2 * TBLK, 128), jnp.int32),
        ] + [pltpu.VMEM((128, 32), jnp.float32)] * RING
          + [pltpu.SemaphoreType.DMA] * RING,
        compiler_params=pltpu.CompilerParams(use_tc_tiling_on_sc=False),
    )
    def k(h_hbm, e_hbm, zz_hbm, out_hbm, acc, eidx, r0, r1, r2, r3,
          g0, g1, g2, g3):
        rows = [r0, r1, r2, r3]
        gsems = [g0, g1, g2, g3]
        c = lax.axis_index("c")
        s = lax.axis_index("s")
        w0 = s * SUBSL
        pltpu.sync_copy(zz_hbm.at[pl.ds(w0, SUBSL)], acc.at[pl.ds(w0, SUBSL)])
        plsc.subcore_barrier()

        def make_body(h_view):
            def body(kk, carry):
                base = (s * 10 + kk) * (2 * TBLK)
                _sc_edge_loop(h_view, e_hbm, acc, eidx, rows, gsems, base)
                return carry
            return body

        @pl.when(c == 0)
        def _():
            lax.fori_loop(0, 10, make_body(h_hbm.at[0]), 0)

        @pl.when(c == 1)
        def _():
            lax.fori_loop(0, 10, make_body(h_hbm.at[1]), 0)

        plsc.subcore_barrier()

        @pl.when(c == 0)
        def _():
            pltpu.sync_copy(acc.at[pl.ds(w0, SUBSL)],
                            out_hbm.at[0].at[pl.ds(w0, SUBSL)])

        @pl.when(c == 1)
        def _():
            pltpu.sync_copy(acc.at[pl.ds(w0, SUBSL)],
                            out_hbm.at[1].at[pl.ds(w0, SUBSL)])

    return k(h_split, e2, zeros32)


def _tc1a_body(aggp_ref, x_ref, a_ref, parts_ref):
    i = pl.program_id(0)
    a = aggp_ref[0] + aggp_ref[1]
    a_ref[...] = a
    x = x_ref[...]
    row = i * BLK + lax.broadcasted_iota(jnp.int32, (BLK, 1), 0)
    mask = row < N
    am = jnp.where(mask, a, 0.0)
    xm = jnp.where(mask, x, 0.0)

    @pl.when(i == 0)
    def _():
        parts_ref[...] = jnp.zeros_like(parts_ref)

    parts_ref[:, 0:1] += am
    parts_ref[:, 1:2] += am * am
    parts_ref[:, 2:3] += xm
    parts_ref[:, 3:4] += xm * xm
    parts_ref[:, 4:5] += am * xm


def _tc1b_body(a_ref, x_ref, parts_ref, al_ref, ber_ref, bb_ref, g_ref,
               bt_ref, h_ref):
    p = parts_ref[...]
    ninv = 1.0 / N
    ma = jnp.sum(p[:, 0]) * ninv
    saa = jnp.sum(p[:, 1]) * ninv
    mx = jnp.sum(p[:, 2]) * ninv
    sxx = jnp.sum(p[:, 3]) * ninv
    sax = jnp.sum(p[:, 4]) * ninv
    va = saa - ma * ma
    vx = sxx - mx * mx
    cax = sax - ma * mx
    al = al_ref[...]
    ber = ber_ref[...]
    bb = bb_ref[...]
    m = ma * al + mx * ber + bb
    v = al * al * va + 2.0 * al * ber * cax + ber * ber * vx
    scale = g_ref[...] * lax.rsqrt(v + EPS)
    A = al * scale
    B = ber * scale
    C = (bb - m) * scale + bt_ref[...]
    a = a_ref[...]
    x = x_ref[...]
    h = a[None, :, :] * A[:, None, :] + x[None, :, :] * B[:, None, :] \
        + C[:, None, :]
    h_ref[...] = jnp.maximum(h, 0.0)


def _tc_layer_body(agg_ref, h_ref, wr_ref, wo_ref, b_ref, z_ref, parts_ref):
    i = pl.program_id(0)
    a = jnp.concatenate([agg_ref[0], agg_ref[1]], axis=1)
    h = jnp.concatenate([h_ref[0], h_ref[1]], axis=1)
    z = (jnp.dot(a, wr_ref[...], preferred_element_type=jnp.float32,
             precision=lax.Precision.HIGHEST)
         + jnp.dot(h, wo_ref[...], preferred_element_type=jnp.float32,
               precision=lax.Precision.HIGHEST)
         + b_ref[...])
    zs = jnp.stack([z[:, :32], z[:, 32:]])
    z_ref[...] = zs
    row = i * BLK + lax.broadcasted_iota(jnp.int32, (1, BLK, 1), 1)
    mask = row < N
    zm = jnp.where(mask, zs, 0.0)

    @pl.when(i == 0)
    def _():
        parts_ref[...] = jnp.zeros_like(parts_ref)

    parts_ref[0] += jnp.sum(zm, axis=1)
    parts_ref[1] += jnp.sum(zm * zs, axis=1)


def _tc_bn_relu_body(z_ref, parts_ref, g_ref, bt_ref, h_ref):
    m = parts_ref[0] * (1.0 / N)
    v = parts_ref[1] * (1.0 / N) - m * m
    scale = g_ref[...] * lax.rsqrt(v + EPS)
    shift = bt_ref[...] - m * scale
    h_ref[...] = jnp.maximum(
        z_ref[...] * scale[:, None, :] + shift[:, None, :], 0.0)


def _tc3b_body(z_ref, parts_ref, g_ref, bt_ref, batch_ref, w_ref, bo_ref,
               out_ref, acc_ref):
    i = pl.program_id(0)
    m = parts_ref[0] * (1.0 / N)
    v = parts_ref[1] * (1.0 / N) - m * m
    scale = g_ref[...] * lax.rsqrt(v + EPS)
    shift = bt_ref[...] - m * scale
    h3 = z_ref[...] * scale[:, None, :] + shift[:, None, :]
    t = h3 * w_ref[...][:, None, :]
    sfull = jnp.sum(t, axis=0)
    s = jnp.sum(sfull, axis=1, keepdims=True)
    b = batch_ref[...]
    oh = (b == lax.broadcasted_iota(jnp.int32, (1, G), 1)).astype(jnp.float32)

    @pl.when(i == 0)
    def _():
        acc_ref[...] = jnp.zeros_like(acc_ref)

    acc_ref[0:1] += jnp.sum(oh * s, axis=0, keepdims=True)
    acc_ref[1:2] += jnp.sum(oh, axis=0, keepdims=True)

    @pl.when(i == NUM_BLK - 1)
    def _():
        ps = acc_ref[0:1]
        cnt = jnp.maximum(acc_ref[1:2], 1.0)
        out_ref[...] = ps / cnt + bo_ref[0, 0]


def _node_spec(rank3=False):
    if rank3:
        return pl.BlockSpec((2, BLK, 32), lambda i: (0, i, 0))
    return pl.BlockSpec((BLK, 1), lambda i: (i, 0))


def _full_spec(shape):
    nd = len(shape)
    return pl.BlockSpec(shape, lambda i: (0,) * nd)


def kernel(x, edge_index, batch, W_rel1, b_rel1, W_root1, gamma1, beta1,
           W_rel2, b_rel2, W_root2, gamma2, beta2, W_rel3, b_rel3, W_root3,
           gamma3, beta3, W_out, b_out):
    f32 = jnp.float32
    src = edge_index[0]
    dst = edge_index[1]
    pad_e = E_ROWS * 128 - E
    src2d = jnp.concatenate(
        [src, jnp.full((pad_e,), DUMMY, jnp.int32)]).reshape(E_ROWS, 128)
    dst2d = jnp.concatenate(
        [dst, jnp.full((pad_e,), DUMMY, jnp.int32)]).reshape(E_ROWS, 128)
    sh = src2d.reshape(16, 10, TBLK, 128)
    dh = dst2d.reshape(16, 10, TBLK, 128)
    e2h = jnp.concatenate([sh, dh], axis=2).reshape(-1, 128)
    s1 = src2d.reshape(2, 16, 5, TBLK, 128)
    d1 = dst2d.reshape(2, 16, 5, TBLK, 128)
    e21 = jnp.concatenate([s1, d1], axis=3).reshape(-1, 128)
    x_pad = jnp.pad(x, ((0, N_PAD - N), (0, 0)))
    x_pad16 = jnp.pad(x, ((0, N_PAD - N), (0, 15)))
    batch2d = jnp.pad(batch, (0, N_PAD - N),
                      constant_values=G).reshape(N_PAD, 1)
    zeros16 = jnp.zeros((N_PAD, 16), f32)
    zeros32 = jnp.zeros((N_PAD, 32), f32)

    al1 = W_rel1[:, 0].reshape(2, 32)
    ber1 = W_root1[:, 0].reshape(2, 32)
    bb1 = b_rel1.reshape(2, 32)
    g1 = gamma1.reshape(2, 32)
    bt1 = beta1.reshape(2, 32)
    g2 = gamma2.reshape(2, 32)
    bt2 = beta2.reshape(2, 32)
    g3 = gamma3.reshape(2, 32)
    bt3 = beta3.reshape(2, 32)
    wr2 = W_rel2.T
    wo2 = W_root2.T
    br2 = b_rel2.reshape(1, 64)
    wr3 = W_rel3.T
    wo3 = W_root3.T
    br3 = b_rel3.reshape(1, 64)
    wout = W_out[0].reshape(2, 32)
    bo = b_out.reshape(1, 1)

    agg1p = _sc_agg_scalar(x_pad16, e21, zeros16)[:, :, 0:1]

    a, parts1 = pl.pallas_call(
        _tc1a_body,
        grid=(NUM_BLK,),
        in_specs=[pl.BlockSpec((2, BLK, 1), lambda i: (0, i, 0)),
                  _node_spec()],
        out_specs=[_node_spec(), _full_spec((BLK, 8))],
        out_shape=[jax.ShapeDtypeStruct((N_PAD, 1), f32),
                   jax.ShapeDtypeStruct((BLK, 8), f32)],
    )(agg1p, x_pad)

    h1 = pl.pallas_call(
        _tc1b_body,
        grid=(NUM_BLK,),
        in_specs=[_node_spec(), _node_spec(), _full_spec((BLK, 8)),
                  _full_spec((2, 32)), _full_spec((2, 32)),
                  _full_spec((2, 32)), _full_spec((2, 32)),
                  _full_spec((2, 32))],
        out_specs=_node_spec(rank3=True),
        out_shape=jax.ShapeDtypeStruct((2, N_PAD, 32), f32),
    )(a, x_pad, parts1, al1, ber1, bb1, g1, bt1)

    agg2 = _sc_agg_h(h1, e2h, zeros32)

    z2, parts2 = pl.pallas_call(
        _tc_layer_body,
        grid=(NUM_BLK,),
        in_specs=[_node_spec(rank3=True), _node_spec(rank3=True),
                  _full_spec((64, 64)), _full_spec((64, 64)),
                  _full_spec((1, 64))],
        out_specs=[_node_spec(rank3=True), _full_spec((2, 2, 32))],
        out_shape=[jax.ShapeDtypeStruct((2, N_PAD, 32), f32),
                   jax.ShapeDtypeStruct((2, 2, 32), f32)],
    )(agg2, h1, wr2, wo2, br2)

    h2 = pl.pallas_call(
        _tc_bn_relu_body,
        grid=(NUM_BLK,),
        in_specs=[_node_spec(rank3=True), _full_spec((2, 2, 32)),
                  _full_spec((2, 32)), _full_spec((2, 32))],
        out_specs=_node_spec(rank3=True),
        out_shape=jax.ShapeDtypeStruct((2, N_PAD, 32), f32),
    )(z2, parts2, g2, bt2)

    agg3 = _sc_agg_h(h2, e2h, zeros32)

    z3, parts3 = pl.pallas_call(
        _tc_layer_body,
        grid=(NUM_BLK,),
        in_specs=[_node_spec(rank3=True), _node_spec(rank3=True),
                  _full_spec((64, 64)), _full_spec((64, 64)),
                  _full_spec((1, 64))],
        out_specs=[_node_spec(rank3=True), _full_spec((2, 2, 32))],
        out_shape=[jax.ShapeDtypeStruct((2, N_PAD, 32), f32),
                   jax.ShapeDtypeStruct((2, 2, 32), f32)],
    )(agg3, h2, wr3, wo3, br3)

    out = pl.pallas_call(
        _tc3b_body,
        grid=(NUM_BLK,),
        in_specs=[_node_spec(rank3=True), _full_spec((2, 2, 32)),
                  _full_spec((2, 32)), _full_spec((2, 32)),
                  _node_spec(), _full_spec((2, 32)), _full_spec((1, 1))],
        out_specs=_full_spec((1, G)),
        out_shape=jax.ShapeDtypeStruct((1, G), f32),
        scratch_shapes=[pltpu.VMEM((2, G), f32)],
    )(z3, parts3, g3, bt3, batch2d, wout, bo)

    return out.reshape(G, 1)
